# baseline (speedup 1.0000x reference)
import functools

import jax
import jax.numpy as jnp
from jax import lax
from jax.experimental import pallas as pl
from jax.experimental.pallas import tpu as pltpu
from jax.experimental.pallas import tpu_sc as plsc

_CH = 512


def _phase1_kernel(B, V, D, NC, NS):
    NW = NC * NS
    n_ck_total = V // _CH
    aligned_v = n_ck_total * _CH
    n_extra = n_ck_total - (n_ck_total // NW) * NW
    base_ck = n_ck_total // NW
    mesh = plsc.VectorSubcoreMesh(core_axis_name="c", subcore_axis_name="s")

    @functools.partial(
        pl.kernel,
        mesh=mesh,
        compiler_params=pltpu.CompilerParams(needs_layout_passes=False),
        out_type=jax.ShapeDtypeStruct((B * D,), jnp.float32),
        scratch_types=[
            pltpu.VMEM((B,), jnp.int32),
            pltpu.VMEM((B + 16,), jnp.int32),
            pltpu.VMEM((B + 16,), jnp.int32),
            pltpu.VMEM((D, _CH), jnp.float32),
            pltpu.VMEM((16, D), jnp.float32),
            pltpu.SemaphoreType.DMA,
        ],
    )
    def k(idx_hbm, tT_hbm, tail_hbm, packed_hbm, idx_v, hitpos_v, hitval_v,
          chunk_v, rowb_v, sem_r):
        wid = lax.axis_index("s") * NC + lax.axis_index("c")
        n_ck = base_ck + jnp.where(wid < n_extra, 1, 0)
        start_ck = wid * base_ck + jnp.minimum(wid, n_extra)
        lo = start_ck * _CH
        hi = lo + n_ck * _CH
        is_last = wid == NW - 1
        sel_hi = jnp.where(is_last, V, hi)

        pltpu.sync_copy(idx_hbm, idx_v)
        iota = lax.iota(jnp.int32, 16)

        def sel_body(g, cnt):
            v = idx_v[pl.ds(g * 16, 16)]
            m = (v >= lo) & (v < sel_hi)
            pos = g * 16 + iota
            plsc.store_compressed(hitpos_v.at[pl.ds(cnt, 16)], pos, mask=m)
            plsc.store_compressed(hitval_v.at[pl.ds(cnt, 16)], v, mask=m)
            return cnt + plsc.all_reduce_population_count(m)[0]

        cnt = lax.fori_loop(0, B // 16, sel_body, jnp.int32(0))
        hitval_v[pl.ds(cnt, 16)] = jnp.full((16,), -1, jnp.int32)
        n_grp = (cnt + 15) // 16

        def ck_body(c, n_prev):
            off = lo + c * _CH
            pltpu.sync_copy(tT_hbm.at[:, pl.ds(off, _CH)], chunk_v)

            def scan_body(hg, carry):
                n_out = carry
                def drain(_, x):
                    pltpu.make_async_copy(
                        rowb_v.at[0], packed_hbm.at[pl.ds(0, D)], sem_r
                    ).wait()
                    return x
                lax.fori_loop(0, n_out, drain, 0)
                hv = hitval_v[pl.ds(hg * 16, 16)]
                hp = hitpos_v[pl.ds(hg * 16, 16)]
                m2 = (hv >= off) & (hv < off + _CH)
                m2i = m2.astype(jnp.int32)
                for e in range(16):
                    @pl.when(m2i[e] != 0)
                    def _():
                        i = jnp.broadcast_to(hv[e] - off, (16,))
                        for q in range(D // 16):
                            col = plsc.load_gather(
                                chunk_v, [iota + q * 16, i]
                            )
                            rowb_v[e, pl.ds(q * 16, 16)] = col
                        pltpu.async_copy(
                            rowb_v.at[e],
                            packed_hbm.at[pl.ds(hp[e] * D, D)],
                            sem_r,
                        )
                return plsc.all_reduce_population_count(m2)[0]

            n_last = lax.fori_loop(0, n_grp, scan_body, n_prev)
            return n_last

        n_out = lax.fori_loop(0, n_ck, ck_body, jnp.int32(0))

        def drain_tail(_, x):
            pltpu.make_async_copy(
                rowb_v.at[0], packed_hbm.at[pl.ds(0, D)], sem_r
            ).wait()
            return x
        lax.fori_loop(0, n_out, drain_tail, 0)

        @pl.when(is_last)
        def _():
            def tail_body(hg, carry):
                hv = hitval_v[pl.ds(hg * 16, 16)]
                hp = hitpos_v[pl.ds(hg * 16, 16)]
                m3i = (hv >= aligned_v).astype(jnp.int32)
                for e in range(16):
                    @pl.when(m3i[e] != 0)
                    def _():
                        pltpu.make_async_copy(
                            tail_hbm.at[pl.ds((hv[e] - aligned_v) * D, D)],
                            rowb_v.at[e],
                            sem_r,
                        ).start()
                        pltpu.make_async_copy(
                            tail_hbm.at[pl.ds(0, D)], rowb_v.at[e], sem_r
                        ).wait()
                        pltpu.async_copy(
                            rowb_v.at[e],
                            packed_hbm.at[pl.ds(hp[e] * D, D)],
                            sem_r,
                        ).wait()
                return carry
            lax.fori_loop(0, n_grp, tail_body, 0)

    return k


def _phase2_kernel(B, D, NC, NS):
    NW = NC * NS
    bw = B // NW
    mesh = plsc.VectorSubcoreMesh(core_axis_name="c", subcore_axis_name="s")

    @functools.partial(
        pl.kernel,
        mesh=mesh,
        compiler_params=pltpu.CompilerParams(needs_layout_passes=False),
        out_type=jax.ShapeDtypeStruct((D, B), jnp.float32),
        scratch_types=[
            pltpu.VMEM((bw * D,), jnp.float32),
            pltpu.VMEM((D, bw), jnp.float32),
        ],
    )
    def k(packed_hbm, outT_hbm, pk_v, out_v):
        wid = lax.axis_index("s") * NC + lax.axis_index("c")
        base = wid * bw
        pltpu.sync_copy(packed_hbm.at[pl.ds(base * D, bw * D)], pk_v)
        iota = lax.iota(jnp.int32, 16)

        def eg_body(eg, carry):
            ids = (iota + eg * 16) * D
            for j in range(D):
                out_v[j, pl.ds(eg * 16, 16)] = plsc.load_gather(
                    pk_v, [ids + j]
                )
            return carry

        lax.fori_loop(0, bw // 16, eg_body, 0)
        pltpu.sync_copy(out_v, outT_hbm.at[:, pl.ds(base, bw)])

    return k


def kernel(indices, codes_weight):
    if indices.ndim > 1:
        indices = jnp.squeeze(indices, axis=-1)
    B = indices.shape[0]
    V, D = codes_weight.shape
    info = plsc.get_sparse_core_info()
    NC, NS = info.num_cores, info.num_subcores
    idx = indices.astype(jnp.int32)
    tT = codes_weight.T
    aligned_v = (V // _CH) * _CH
    tail_flat = codes_weight[aligned_v:, :].reshape(-1)
    packed = _phase1_kernel(B, V, D, NC, NS)(idx, tT, tail_flat)
    outT = _phase2_kernel(B, D, NC, NS)(packed)
    return outT.T


# speedup vs baseline: 6.1228x; 6.1228x over previous
import functools

import jax
import jax.numpy as jnp
from jax import lax
from jax.experimental import pallas as pl
from jax.experimental.pallas import tpu as pltpu
from jax.experimental.pallas import tpu_sc as plsc

_CH = 256
_KCAP = 32
_NROWB = 32
_MAXOUT = 24


def _phase1_kernel(B, V, D, NC, NS):
    NW = NC * NS
    n_ck_total = V // _CH
    aligned_v = n_ck_total * _CH
    base_ck = n_ck_total // NW
    n_extra = n_ck_total - base_ck * NW
    max_ck = base_ck + 1
    mesh = plsc.VectorSubcoreMesh(core_axis_name="c", subcore_axis_name="s")

    @functools.partial(
        pl.kernel,
        mesh=mesh,
        compiler_params=pltpu.CompilerParams(needs_layout_passes=False),
        out_type=jax.ShapeDtypeStruct((B * D,), jnp.float32),
        scratch_types=[
            pltpu.VMEM((B,), jnp.int32),
            pltpu.VMEM((B + 16,), jnp.int32),
            pltpu.VMEM((B + 16,), jnp.int32),
            pltpu.VMEM((max_ck * _KCAP + 16,), jnp.int32),
            pltpu.VMEM((2, D, _CH), jnp.float32),
            pltpu.VMEM((_NROWB, D), jnp.float32),
            pltpu.SMEM((max_ck + 8,), jnp.int32),
            pltpu.SemaphoreType.DMA,
            pltpu.SemaphoreType.DMA,
            pltpu.SemaphoreType.DMA,
        ],
    )
    def k(idx_hbm, tT_hbm, tail_hbm, packed_hbm, idx_v, hitpos_v, hitval_v,
          bk_v, chunk_v, rowb_v, cnt_s, sem_r, sem_c0, sem_c1):
        wid = lax.axis_index("s") * NC + lax.axis_index("c")
        n_ck = base_ck + jnp.where(wid < n_extra, 1, 0)
        start_ck = wid * base_ck + jnp.minimum(wid, n_extra)
        lo = start_ck * _CH
        hi = lo + n_ck * _CH
        is_last = wid == NW - 1
        sel_hi = jnp.where(is_last, V, hi)
        of_slot = max_ck + 1

        pltpu.sync_copy(idx_hbm, idx_v)
        iota = lax.iota(jnp.int32, 16)

        def sel_body(g, cnt):
            v = idx_v[pl.ds(g * 16, 16)]
            m = (v >= lo) & (v < sel_hi)
            pos = g * 16 + iota
            plsc.store_compressed(hitpos_v.at[pl.ds(cnt, 16)], pos, mask=m)
            plsc.store_compressed(hitval_v.at[pl.ds(cnt, 16)], v, mask=m)
            return cnt + plsc.all_reduce_population_count(m)[0]

        cnt = lax.fori_loop(0, B // 16, sel_body, jnp.int32(0))
        hitval_v[pl.ds(cnt, 16)] = jnp.full((16,), -1, jnp.int32)
        n_grp = (cnt + 15) // 16

        def zero_body(c, x):
            cnt_s[c] = jnp.int32(0)
            return x
        lax.fori_loop(0, max_ck + 8, zero_body, 0)

        one_lane = iota == 0

        def bk_body(hg, x):
            hv = hitval_v[pl.ds(hg * 16, 16)]
            hp = hitpos_v[pl.ds(hg * 16, 16)]
            mi = ((hv >= lo) & (hv < hi)).astype(jnp.int32)
            for e in range(16):
                @pl.when(mi[e] != 0)
                def _():
                    rel = hv[e] - lo
                    c = rel // _CH
                    slot = cnt_s[c]
                    cnt_s[c] = slot + 1

                    @pl.when(slot < _KCAP)
                    def _():
                        entry = (hp[e] * 256) + (rel % _CH)
                        plsc.store_compressed(
                            bk_v.at[pl.ds(c * _KCAP + slot, 16)],
                            jnp.broadcast_to(entry, (16,)),
                            mask=one_lane,
                        )

                    @pl.when(slot >= _KCAP)
                    def _():
                        cnt_s[of_slot] = jnp.int32(1)
            return x

        lax.fori_loop(0, n_grp, bk_body, 0)
        overflow = cnt_s[of_slot]

        def drain_one(_, x):
            pltpu.make_async_copy(
                rowb_v.at[0], packed_hbm.at[pl.ds(0, D)], sem_r
            ).wait()
            return x

        def extract(buf, entry, rb):
            pos = entry // 256
            off = jnp.broadcast_to(entry % 256, (16,))
            bufi = jnp.broadcast_to(buf, (16,))
            for q in range(D // 16):
                col = plsc.load_gather(
                    chunk_v, [bufi, iota + q * 16, off]
                )
                rowb_v[rb, pl.ds(q * 16, 16)] = col
            pltpu.async_copy(
                rowb_v.at[rb], packed_hbm.at[pl.ds(pos * D, D)], sem_r
            )

        pltpu.make_async_copy(
            tT_hbm.at[:, pl.ds(lo, _CH)], chunk_v.at[0], sem_c0
        ).start()

        def ck_body(c, n_prev):
            buf = lax.rem(c, 2)
            nxt = 1 - buf
            @pl.when((c + 1 < n_ck) & (nxt == 1))
            def _():
                pltpu.make_async_copy(
                    tT_hbm.at[:, pl.ds(lo + (c + 1) * _CH, _CH)],
                    chunk_v.at[1],
                    sem_c1,
                ).start()

            @pl.when((c + 1 < n_ck) & (nxt == 0))
            def _():
                pltpu.make_async_copy(
                    tT_hbm.at[:, pl.ds(lo + (c + 1) * _CH, _CH)],
                    chunk_v.at[0],
                    sem_c0,
                ).start()

            lax.fori_loop(0, n_prev, drain_one, 0)

            @pl.when(buf == 0)
            def _():
                pltpu.make_async_copy(
                    tT_hbm.at[:, pl.ds(lo + c * _CH, _CH)],
                    chunk_v.at[0],
                    sem_c0,
                ).wait()

            @pl.when(buf == 1)
            def _():
                pltpu.make_async_copy(
                    tT_hbm.at[:, pl.ds(lo + c * _CH, _CH)],
                    chunk_v.at[1],
                    sem_c1,
                ).wait()

            def fast(x):
                nb = jnp.minimum(cnt_s[c], _KCAP)

                def hit_body(t, y):
                    entry = bk_v[pl.ds(c * _KCAP + t, 16)][0]
                    extract(buf, entry, t)
                    return y

                lax.fori_loop(0, nb, hit_body, 0)
                return nb

            def slow(x):
                off = lo + c * _CH

                def grp_body(hg, n_pg):
                    lax.fori_loop(0, n_pg, drain_one, 0)
                    hv = hitval_v[pl.ds(hg * 16, 16)]
                    hp = hitpos_v[pl.ds(hg * 16, 16)]
                    mi = ((hv >= off) & (hv < off + _CH)).astype(jnp.int32)
                    for e in range(16):
                        @pl.when(mi[e] != 0)
                        def _():
                            entry = hp[e] * 256 + (hv[e] - off)
                            extract(buf, entry, e)
                    return plsc.all_reduce_population_count(mi != 0)[0]

                return lax.fori_loop(0, n_grp, grp_body, jnp.int32(0))

            return lax.cond(overflow == 0, fast, slow, 0)

        n_out = lax.fori_loop(0, n_ck, ck_body, jnp.int32(0))
        lax.fori_loop(0, n_out, drain_one, 0)

        @pl.when(is_last)
        def _():
            def tail_body(hg, carry):
                hv = hitval_v[pl.ds(hg * 16, 16)]
                hp = hitpos_v[pl.ds(hg * 16, 16)]
                m3i = (hv >= aligned_v).astype(jnp.int32)
                for e in range(16):
                    @pl.when(m3i[e] != 0)
                    def _():
                        pltpu.make_async_copy(
                            tail_hbm.at[pl.ds((hv[e] - aligned_v) * D, D)],
                            rowb_v.at[0],
                            sem_r,
                        ).start()
                        pltpu.make_async_copy(
                            tail_hbm.at[pl.ds(0, D)], rowb_v.at[0], sem_r
                        ).wait()
                        pltpu.async_copy(
                            rowb_v.at[0],
                            packed_hbm.at[pl.ds(hp[e] * D, D)],
                            sem_r,
                        ).wait()
                return carry
            lax.fori_loop(0, n_grp, tail_body, 0)

    return k


def _phase2_kernel(B, D, NC, NS):
    NW = NC * NS
    bw = B // NW
    mesh = plsc.VectorSubcoreMesh(core_axis_name="c", subcore_axis_name="s")

    @functools.partial(
        pl.kernel,
        mesh=mesh,
        compiler_params=pltpu.CompilerParams(needs_layout_passes=False),
        out_type=jax.ShapeDtypeStruct((D, B), jnp.float32),
        scratch_types=[
            pltpu.VMEM((bw * D,), jnp.float32),
            pltpu.VMEM((D, bw), jnp.float32),
        ],
    )
    def k(packed_hbm, outT_hbm, pk_v, out_v):
        wid = lax.axis_index("s") * NC + lax.axis_index("c")
        base = wid * bw
        pltpu.sync_copy(packed_hbm.at[pl.ds(base * D, bw * D)], pk_v)
        iota = lax.iota(jnp.int32, 16)

        def eg_body(eg, carry):
            ids = (iota + eg * 16) * D
            for j in range(D):
                out_v[j, pl.ds(eg * 16, 16)] = plsc.load_gather(
                    pk_v, [ids + j]
                )
            return carry

        lax.fori_loop(0, bw // 16, eg_body, 0)
        pltpu.sync_copy(out_v, outT_hbm.at[:, pl.ds(base, bw)])

    return k


def kernel(indices, codes_weight):
    if indices.ndim > 1:
        indices = jnp.squeeze(indices, axis=-1)
    B = indices.shape[0]
    V, D = codes_weight.shape
    info = plsc.get_sparse_core_info()
    NC, NS = info.num_cores, info.num_subcores
    idx = indices.astype(jnp.int32)
    tT = codes_weight.T
    aligned_v = (V // _CH) * _CH
    tail_flat = codes_weight[aligned_v:, :].reshape(-1)
    packed = _phase1_kernel(B, V, D, NC, NS)(idx, tT, tail_flat)
    outT = _phase2_kernel(B, D, NC, NS)(packed)
    return outT.T


# speedup vs baseline: 6.4208x; 1.0487x over previous
import functools

import jax
import jax.numpy as jnp
from jax import lax
from jax.experimental import pallas as pl
from jax.experimental.pallas import tpu as pltpu
from jax.experimental.pallas import tpu_sc as plsc

_CH = 512
_KCAP = 32
_NROWB = 64


def _phase1_kernel(B, V, D, NC, NS):
    NW = NC * NS
    n_ck_total = V // _CH
    aligned_v = n_ck_total * _CH
    base_ck = n_ck_total // NW
    n_extra = n_ck_total - base_ck * NW
    max_ck = base_ck + 1
    mesh = plsc.VectorSubcoreMesh(core_axis_name="c", subcore_axis_name="s")

    @functools.partial(
        pl.kernel,
        mesh=mesh,
        compiler_params=pltpu.CompilerParams(needs_layout_passes=False),
        out_type=jax.ShapeDtypeStruct((B * D,), jnp.float32),
        scratch_types=[
            pltpu.VMEM((B,), jnp.int32),
            pltpu.VMEM((B + 16,), jnp.int32),
            pltpu.VMEM((B + 16,), jnp.int32),
            pltpu.VMEM((max_ck * _KCAP + 16,), jnp.int32),
            pltpu.VMEM((2, D, _CH), jnp.float32),
            pltpu.VMEM((_NROWB, D), jnp.float32),
            pltpu.SMEM((max_ck + 8,), jnp.int32),
            pltpu.SemaphoreType.DMA,
            pltpu.SemaphoreType.DMA,
            pltpu.SemaphoreType.DMA,
        ],
    )
    def k(idx_hbm, tT_hbm, tail_hbm, packed_hbm, idx_v, hitpos_v, hitval_v,
          bk_v, chunk_v, rowb_v, cnt_s, sem_r, sem_c0, sem_c1):
        wid = lax.axis_index("s") * NC + lax.axis_index("c")
        n_ck = base_ck + jnp.where(wid < n_extra, 1, 0)
        start_ck = wid * base_ck + jnp.minimum(wid, n_extra)
        lo = start_ck * _CH
        hi = lo + n_ck * _CH
        is_last = wid == NW - 1
        sel_hi = jnp.where(is_last, V, hi)
        of_slot = max_ck + 1

        pltpu.sync_copy(idx_hbm, idx_v)
        iota = lax.iota(jnp.int32, 16)

        def sel_body(g, cnt):
            v = idx_v[pl.ds(g * 16, 16)]
            m = (v >= lo) & (v < sel_hi)
            pos = g * 16 + iota
            plsc.store_compressed(hitpos_v.at[pl.ds(cnt, 16)], pos, mask=m)
            plsc.store_compressed(hitval_v.at[pl.ds(cnt, 16)], v, mask=m)
            return cnt + plsc.all_reduce_population_count(m)[0]

        cnt = lax.fori_loop(0, B // 16, sel_body, jnp.int32(0))
        hitval_v[pl.ds(cnt, 16)] = jnp.full((16,), -1, jnp.int32)
        n_grp = (cnt + 15) // 16

        def zero_body(c, x):
            cnt_s[c] = jnp.int32(0)
            return x
        lax.fori_loop(0, max_ck + 8, zero_body, 0)

        one_lane = iota == 0

        def bk_body(hg, x):
            hv = hitval_v[pl.ds(hg * 16, 16)]
            hp = hitpos_v[pl.ds(hg * 16, 16)]
            mi = ((hv >= lo) & (hv < hi)).astype(jnp.int32)
            for e in range(16):
                @pl.when(mi[e] != 0)
                def _():
                    rel = hv[e] - lo
                    c = rel // _CH
                    slot = cnt_s[c]
                    cnt_s[c] = slot + 1

                    @pl.when(slot < _KCAP)
                    def _():
                        entry = (hp[e] * 1024) + (rel % _CH)
                        plsc.store_compressed(
                            bk_v.at[pl.ds(c * _KCAP + slot, 16)],
                            jnp.broadcast_to(entry, (16,)),
                            mask=one_lane,
                        )

                    @pl.when(slot >= _KCAP)
                    def _():
                        cnt_s[of_slot] = jnp.int32(1)
            return x

        lax.fori_loop(0, n_grp, bk_body, 0)
        overflow = cnt_s[of_slot]

        def drain_one(_, x):
            pltpu.make_async_copy(
                rowb_v.at[0], packed_hbm.at[pl.ds(0, D)], sem_r
            ).wait()
            return x

        def extract(buf, entry, rb):
            pos = entry // 1024
            off = jnp.broadcast_to(entry % 1024, (16,))
            bufi = jnp.broadcast_to(buf, (16,))
            for q in range(D // 16):
                col = plsc.load_gather(
                    chunk_v, [bufi, iota + q * 16, off]
                )
                rowb_v[rb, pl.ds(q * 16, 16)] = col
            pltpu.async_copy(
                rowb_v.at[rb], packed_hbm.at[pl.ds(pos * D, D)], sem_r
            )

        pltpu.make_async_copy(
            tT_hbm.at[:, pl.ds(lo, _CH)], chunk_v.at[0], sem_c0
        ).start()

        def ck_body(c, n_prev):
            buf = lax.rem(c, 2)
            nxt = 1 - buf
            @pl.when((c + 1 < n_ck) & (nxt == 1))
            def _():
                pltpu.make_async_copy(
                    tT_hbm.at[:, pl.ds(lo + (c + 1) * _CH, _CH)],
                    chunk_v.at[1],
                    sem_c1,
                ).start()

            @pl.when((c + 1 < n_ck) & (nxt == 0))
            def _():
                pltpu.make_async_copy(
                    tT_hbm.at[:, pl.ds(lo + (c + 1) * _CH, _CH)],
                    chunk_v.at[0],
                    sem_c0,
                ).start()

            @pl.when(buf == 0)
            def _():
                pltpu.make_async_copy(
                    tT_hbm.at[:, pl.ds(lo + c * _CH, _CH)],
                    chunk_v.at[0],
                    sem_c0,
                ).wait()

            @pl.when(buf == 1)
            def _():
                pltpu.make_async_copy(
                    tT_hbm.at[:, pl.ds(lo + c * _CH, _CH)],
                    chunk_v.at[1],
                    sem_c1,
                ).wait()

            def fast(x):
                nb = jnp.minimum(cnt_s[c], _KCAP)
                sbase = buf * _KCAP

                def hit_body(t, y):
                    entry = bk_v[pl.ds(c * _KCAP + t, 16)][0]
                    extract(buf, entry, sbase + t)
                    return y

                lax.fori_loop(0, nb, hit_body, 0)
                return nb

            def slow(x):
                off = lo + c * _CH
                sbase = buf * _KCAP

                def grp_body(hg, n_pg):
                    lax.fori_loop(0, n_pg, drain_one, 0)
                    hv = hitval_v[pl.ds(hg * 16, 16)]
                    hp = hitpos_v[pl.ds(hg * 16, 16)]
                    mi = ((hv >= off) & (hv < off + _CH)).astype(jnp.int32)
                    for e in range(16):
                        @pl.when(mi[e] != 0)
                        def _():
                            entry = hp[e] * 1024 + (hv[e] - off)
                            extract(buf, entry, sbase + e)
                    return plsc.all_reduce_population_count(mi != 0)[0]

                return lax.fori_loop(0, n_grp, grp_body, jnp.int32(0))

            return lax.cond(overflow == 0, fast, slow, 0)

        def ck_outer(c, carry):
            n1, n2 = carry
            lax.fori_loop(0, n2, drain_one, 0)
            n0 = ck_body(c, jnp.int32(0))
            return (n0, n1)

        n1, n2 = lax.fori_loop(
            0, n_ck, ck_outer, (jnp.int32(0), jnp.int32(0))
        )
        lax.fori_loop(0, n1 + n2, drain_one, 0)

        @pl.when(is_last)
        def _():
            def tail_body(hg, carry):
                hv = hitval_v[pl.ds(hg * 16, 16)]
                hp = hitpos_v[pl.ds(hg * 16, 16)]
                m3i = (hv >= aligned_v).astype(jnp.int32)
                for e in range(16):
                    @pl.when(m3i[e] != 0)
                    def _():
                        pltpu.make_async_copy(
                            tail_hbm.at[pl.ds((hv[e] - aligned_v) * D, D)],
                            rowb_v.at[0],
                            sem_r,
                        ).start()
                        pltpu.make_async_copy(
                            tail_hbm.at[pl.ds(0, D)], rowb_v.at[0], sem_r
                        ).wait()
                        pltpu.async_copy(
                            rowb_v.at[0],
                            packed_hbm.at[pl.ds(hp[e] * D, D)],
                            sem_r,
                        ).wait()
                return carry
            lax.fori_loop(0, n_grp, tail_body, 0)

    return k


def _phase2_kernel(B, D, NC, NS):
    NW = NC * NS
    bw = B // NW
    mesh = plsc.VectorSubcoreMesh(core_axis_name="c", subcore_axis_name="s")

    @functools.partial(
        pl.kernel,
        mesh=mesh,
        compiler_params=pltpu.CompilerParams(needs_layout_passes=False),
        out_type=jax.ShapeDtypeStruct((D, B), jnp.float32),
        scratch_types=[
            pltpu.VMEM((bw * D,), jnp.float32),
            pltpu.VMEM((D, bw), jnp.float32),
        ],
    )
    def k(packed_hbm, outT_hbm, pk_v, out_v):
        wid = lax.axis_index("s") * NC + lax.axis_index("c")
        base = wid * bw
        pltpu.sync_copy(packed_hbm.at[pl.ds(base * D, bw * D)], pk_v)
        iota = lax.iota(jnp.int32, 16)

        def eg_body(eg, carry):
            ids = (iota + eg * 16) * D
            for j in range(D):
                out_v[j, pl.ds(eg * 16, 16)] = plsc.load_gather(
                    pk_v, [ids + j]
                )
            return carry

        lax.fori_loop(0, bw // 16, eg_body, 0)
        pltpu.sync_copy(out_v, outT_hbm.at[:, pl.ds(base, bw)])

    return k


def kernel(indices, codes_weight):
    if indices.ndim > 1:
        indices = jnp.squeeze(indices, axis=-1)
    B = indices.shape[0]
    V, D = codes_weight.shape
    info = plsc.get_sparse_core_info()
    NC, NS = info.num_cores, info.num_subcores
    idx = indices.astype(jnp.int32)
    tT = codes_weight.T
    aligned_v = (V // _CH) * _CH
    tail_flat = codes_weight[aligned_v:, :].reshape(-1)
    packed = _phase1_kernel(B, V, D, NC, NS)(idx, tT, tail_flat)
    outT = _phase2_kernel(B, D, NC, NS)(packed)
    return outT.T


# speedup vs baseline: 6.6066x; 1.0289x over previous
import functools

import jax
import jax.numpy as jnp
from jax import lax
from jax.experimental import pallas as pl
from jax.experimental.pallas import tpu as pltpu
from jax.experimental.pallas import tpu_sc as plsc

_CH = 512
_KCAP = 32
_NROWB = 64


def _phase1_kernel(B, V, D, NC, NS):
    NW = NC * NS
    n_ck_total = V // _CH
    aligned_v = n_ck_total * _CH
    base_ck = n_ck_total // NW
    n_extra = n_ck_total - base_ck * NW
    max_ck = base_ck + 1
    mesh = plsc.VectorSubcoreMesh(core_axis_name="c", subcore_axis_name="s")

    @functools.partial(
        pl.kernel,
        mesh=mesh,
        compiler_params=pltpu.CompilerParams(needs_layout_passes=False),
        out_type=jax.ShapeDtypeStruct((B * D,), jnp.float32),
        scratch_types=[
            pltpu.VMEM((B,), jnp.int32),
            pltpu.VMEM((B + 16,), jnp.int32),
            pltpu.VMEM((B + 16,), jnp.int32),
            pltpu.VMEM((max_ck * _KCAP + 16,), jnp.int32),
            pltpu.VMEM((2, D, _CH), jnp.float32),
            pltpu.VMEM((_NROWB, D), jnp.float32),
            pltpu.SMEM((max_ck + 8,), jnp.int32),
            pltpu.SemaphoreType.DMA,
            pltpu.SemaphoreType.DMA,
            pltpu.SemaphoreType.DMA,
        ],
    )
    def k(idx_hbm, tT_hbm, tail_hbm, packed_hbm, idx_v, hitpos_v, hitval_v,
          bk_v, chunk_v, rowb_v, cnt_s, sem_r, sem_c0, sem_c1):
        wid = lax.axis_index("s") * NC + lax.axis_index("c")
        n_ck = base_ck + jnp.where(wid < n_extra, 1, 0)
        start_ck = wid * base_ck + jnp.minimum(wid, n_extra)
        lo = start_ck * _CH
        hi = lo + n_ck * _CH
        is_last = wid == NW - 1
        sel_hi = jnp.where(is_last, V, hi)
        of_slot = max_ck + 1

        pltpu.make_async_copy(
            tT_hbm.at[:, pl.ds(lo, _CH)], chunk_v.at[0], sem_c0
        ).start()
        pltpu.make_async_copy(
            tT_hbm.at[:, pl.ds(lo + _CH, _CH)], chunk_v.at[1], sem_c1
        ).start()

        pltpu.sync_copy(idx_hbm, idx_v)
        iota = lax.iota(jnp.int32, 16)

        def sel_body(g, cnt):
            v = idx_v[pl.ds(g * 16, 16)]
            m = (v >= lo) & (v < sel_hi)
            pos = g * 16 + iota
            plsc.store_compressed(hitpos_v.at[pl.ds(cnt, 16)], pos, mask=m)
            plsc.store_compressed(hitval_v.at[pl.ds(cnt, 16)], v, mask=m)
            return cnt + plsc.all_reduce_population_count(m)[0]

        cnt = lax.fori_loop(0, B // 16, sel_body, jnp.int32(0))
        hitval_v[pl.ds(cnt, 16)] = jnp.full((16,), -1, jnp.int32)
        n_grp = (cnt + 15) // 16

        def zero_body(c, x):
            cnt_s[c] = jnp.int32(0)
            return x
        lax.fori_loop(0, max_ck + 8, zero_body, 0)

        one_lane = iota == 0

        def bk_body(hg, x):
            hv = hitval_v[pl.ds(hg * 16, 16)]
            hp = hitpos_v[pl.ds(hg * 16, 16)]
            mi = ((hv >= lo) & (hv < hi)).astype(jnp.int32)
            for e in range(16):
                @pl.when(mi[e] != 0)
                def _():
                    rel = hv[e] - lo
                    c = rel // _CH
                    slot = cnt_s[c]
                    cnt_s[c] = slot + 1

                    @pl.when(slot < _KCAP)
                    def _():
                        entry = (hp[e] * 1024) + (rel % _CH)
                        plsc.store_compressed(
                            bk_v.at[pl.ds(c * _KCAP + slot, 16)],
                            jnp.broadcast_to(entry, (16,)),
                            mask=one_lane,
                        )

                    @pl.when(slot >= _KCAP)
                    def _():
                        cnt_s[of_slot] = jnp.int32(1)
            return x

        lax.fori_loop(0, n_grp, bk_body, 0)
        overflow = cnt_s[of_slot]

        def drain_one(_, x):
            pltpu.make_async_copy(
                rowb_v.at[0], packed_hbm.at[pl.ds(0, D)], sem_r
            ).wait()
            return x

        def extract(buf, entry, rb):
            pos = entry // 1024
            off = jnp.broadcast_to(entry % 1024, (16,))
            bufi = jnp.broadcast_to(buf, (16,))
            for q in range(D // 16):
                col = plsc.load_gather(
                    chunk_v, [bufi, iota + q * 16, off]
                )
                rowb_v[rb, pl.ds(q * 16, 16)] = col
            pltpu.async_copy(
                rowb_v.at[rb], packed_hbm.at[pl.ds(pos * D, D)], sem_r
            )

        def ck_body(c, n_prev):
            buf = lax.rem(c, 2)
            @pl.when(buf == 0)
            def _():
                pltpu.make_async_copy(
                    tT_hbm.at[:, pl.ds(lo + c * _CH, _CH)],
                    chunk_v.at[0],
                    sem_c0,
                ).wait()

            @pl.when(buf == 1)
            def _():
                pltpu.make_async_copy(
                    tT_hbm.at[:, pl.ds(lo + c * _CH, _CH)],
                    chunk_v.at[1],
                    sem_c1,
                ).wait()

            def fast(x):
                nb = jnp.minimum(cnt_s[c], _KCAP)
                sbase = buf * _KCAP

                def hit_body(t, y):
                    entry = bk_v[pl.ds(c * _KCAP + t, 16)][0]
                    extract(buf, entry, sbase + t)
                    return y

                lax.fori_loop(0, nb, hit_body, 0)
                return nb

            def slow(x):
                off = lo + c * _CH
                sbase = buf * _KCAP

                def grp_body(hg, n_pg):
                    lax.fori_loop(0, n_pg, drain_one, 0)
                    hv = hitval_v[pl.ds(hg * 16, 16)]
                    hp = hitpos_v[pl.ds(hg * 16, 16)]
                    mi = ((hv >= off) & (hv < off + _CH)).astype(jnp.int32)
                    for e in range(16):
                        @pl.when(mi[e] != 0)
                        def _():
                            entry = hp[e] * 1024 + (hv[e] - off)
                            extract(buf, entry, sbase + e)
                    return plsc.all_reduce_population_count(mi != 0)[0]

                return lax.fori_loop(0, n_grp, grp_body, jnp.int32(0))

            nb = lax.cond(overflow == 0, fast, slow, 0)

            @pl.when((c + 2 < n_ck) & (buf == 0))
            def _():
                pltpu.make_async_copy(
                    tT_hbm.at[:, pl.ds(lo + (c + 2) * _CH, _CH)],
                    chunk_v.at[0],
                    sem_c0,
                ).start()

            @pl.when((c + 2 < n_ck) & (buf == 1))
            def _():
                pltpu.make_async_copy(
                    tT_hbm.at[:, pl.ds(lo + (c + 2) * _CH, _CH)],
                    chunk_v.at[1],
                    sem_c1,
                ).start()

            return nb

        def ck_outer(c, carry):
            n1, n2 = carry
            lax.fori_loop(0, n2, drain_one, 0)
            n0 = ck_body(c, jnp.int32(0))
            return (n0, n1)

        n1, n2 = lax.fori_loop(
            0, n_ck, ck_outer, (jnp.int32(0), jnp.int32(0))
        )
        lax.fori_loop(0, n1 + n2, drain_one, 0)

        @pl.when(is_last)
        def _():
            def tail_body(hg, carry):
                hv = hitval_v[pl.ds(hg * 16, 16)]
                hp = hitpos_v[pl.ds(hg * 16, 16)]
                m3i = (hv >= aligned_v).astype(jnp.int32)
                for e in range(16):
                    @pl.when(m3i[e] != 0)
                    def _():
                        pltpu.make_async_copy(
                            tail_hbm.at[pl.ds((hv[e] - aligned_v) * D, D)],
                            rowb_v.at[0],
                            sem_r,
                        ).start()
                        pltpu.make_async_copy(
                            tail_hbm.at[pl.ds(0, D)], rowb_v.at[0], sem_r
                        ).wait()
                        pltpu.async_copy(
                            rowb_v.at[0],
                            packed_hbm.at[pl.ds(hp[e] * D, D)],
                            sem_r,
                        ).wait()
                return carry
            lax.fori_loop(0, n_grp, tail_body, 0)

    return k


def _phase2_kernel(B, D, NC, NS):
    NW = NC * NS
    bw = B // NW
    mesh = plsc.VectorSubcoreMesh(core_axis_name="c", subcore_axis_name="s")

    oddw = bw + 15

    @functools.partial(
        pl.kernel,
        mesh=mesh,
        compiler_params=pltpu.CompilerParams(needs_layout_passes=False),
        out_type=jax.ShapeDtypeStruct((D, B), jnp.float32),
        scratch_types=[
            pltpu.VMEM((bw * D,), jnp.float32),
            pltpu.VMEM((D, oddw), jnp.float32),
        ],
    )
    def k(packed_hbm, outT_hbm, pk_v, out_v):
        wid = lax.axis_index("s") * NC + lax.axis_index("c")
        base = wid * bw
        pltpu.sync_copy(packed_hbm.at[pl.ds(base * D, bw * D)], pk_v)
        iota = lax.iota(jnp.int32, 16)
        rows_q = [iota + q * 16 for q in range(D // 16)]

        def eg_body(eg, carry):
            for es in range(16):
                e = eg * 16 + es
                col = jnp.broadcast_to(e, (16,))
                rowbase = e * D
                for q in range(D // 16):
                    vals = pk_v[pl.ds(rowbase + q * 16, 16)]
                    plsc.store_scatter(out_v, [rows_q[q], col], vals)
            return carry

        lax.fori_loop(0, bw // 16, eg_body, 0)
        pltpu.sync_copy(
            out_v.at[:, pl.ds(0, bw)], outT_hbm.at[:, pl.ds(base, bw)]
        )

    return k


def kernel(indices, codes_weight):
    if indices.ndim > 1:
        indices = jnp.squeeze(indices, axis=-1)
    B = indices.shape[0]
    V, D = codes_weight.shape
    info = plsc.get_sparse_core_info()
    NC, NS = info.num_cores, info.num_subcores
    idx = indices.astype(jnp.int32)
    tT = codes_weight.T
    aligned_v = (V // _CH) * _CH
    tail_flat = codes_weight[aligned_v:, :].reshape(-1)
    packed = _phase1_kernel(B, V, D, NC, NS)(idx, tT, tail_flat)
    outT = _phase2_kernel(B, D, NC, NS)(packed)
    return outT.T


# speedup vs baseline: 7.0776x; 1.0713x over previous
import functools

import jax
import jax.numpy as jnp
from jax import lax
from jax.experimental import pallas as pl
from jax.experimental.pallas import tpu as pltpu
from jax.experimental.pallas import tpu_sc as plsc

_CH = 512
_KCAP = 32
_NROWB = 64


def _phase1_kernel(B, V, D, NC, NS):
    NW = NC * NS
    n_ck_total = V // _CH
    aligned_v = n_ck_total * _CH
    base_ck = n_ck_total // NW
    n_extra = n_ck_total - base_ck * NW
    max_ck = base_ck + 1
    mesh = plsc.VectorSubcoreMesh(core_axis_name="c", subcore_axis_name="s")

    @functools.partial(
        pl.kernel,
        mesh=mesh,
        compiler_params=pltpu.CompilerParams(needs_layout_passes=False),
        out_type=jax.ShapeDtypeStruct((B * D,), jnp.float32),
        scratch_types=[
            pltpu.VMEM((B,), jnp.int32),
            pltpu.VMEM((B + 16,), jnp.int32),
            pltpu.VMEM((B + 16,), jnp.int32),
            pltpu.VMEM((max_ck * _KCAP + 16,), jnp.int32),
            pltpu.VMEM((2, D, _CH), jnp.float32),
            pltpu.VMEM((_NROWB, D), jnp.float32),
            pltpu.SMEM((max_ck + 8,), jnp.int32),
            pltpu.SemaphoreType.DMA,
            pltpu.SemaphoreType.DMA,
            pltpu.SemaphoreType.DMA,
        ],
    )
    def k(idx_hbm, tT_hbm, tail_hbm, packed_hbm, idx_v, hitpos_v, hitval_v,
          bk_v, chunk_v, rowb_v, cnt_s, sem_r, sem_c0, sem_c1):
        wid = lax.axis_index("s") * NC + lax.axis_index("c")
        n_ck = base_ck + jnp.where(wid < n_extra, 1, 0)
        start_ck = wid * base_ck + jnp.minimum(wid, n_extra)
        lo = start_ck * _CH
        hi = lo + n_ck * _CH
        is_last = wid == NW - 1
        sel_hi = jnp.where(is_last, V, hi)
        of_slot = max_ck + 1

        pltpu.make_async_copy(
            tT_hbm.at[:, pl.ds(lo, _CH)], chunk_v.at[0], sem_c0
        ).start()
        pltpu.make_async_copy(
            tT_hbm.at[:, pl.ds(lo + _CH, _CH)], chunk_v.at[1], sem_c1
        ).start()

        pltpu.sync_copy(idx_hbm, idx_v)
        iota = lax.iota(jnp.int32, 16)

        def sel_body(g, cnt):
            v = idx_v[pl.ds(g * 16, 16)]
            m = (v >= lo) & (v < sel_hi)
            pos = g * 16 + iota
            plsc.store_compressed(hitpos_v.at[pl.ds(cnt, 16)], pos, mask=m)
            plsc.store_compressed(hitval_v.at[pl.ds(cnt, 16)], v, mask=m)
            return cnt + plsc.all_reduce_population_count(m)[0]

        cnt = lax.fori_loop(0, B // 16, sel_body, jnp.int32(0))
        hitval_v[pl.ds(cnt, 16)] = jnp.full((16,), -1, jnp.int32)
        n_grp = (cnt + 15) // 16

        def zero_body(c, x):
            cnt_s[c] = jnp.int32(0)
            return x
        lax.fori_loop(0, max_ck + 8, zero_body, 0)

        one_lane = iota == 0

        def bk_body(hg, x):
            hv = hitval_v[pl.ds(hg * 16, 16)]
            hp = hitpos_v[pl.ds(hg * 16, 16)]
            mi = ((hv >= lo) & (hv < hi)).astype(jnp.int32)
            for e in range(16):
                @pl.when(mi[e] != 0)
                def _():
                    rel = hv[e] - lo
                    c = rel // _CH
                    slot = cnt_s[c]
                    cnt_s[c] = slot + 1

                    @pl.when(slot < _KCAP)
                    def _():
                        entry = (hp[e] * 1024) + (rel % _CH)
                        plsc.store_compressed(
                            bk_v.at[pl.ds(c * _KCAP + slot, 16)],
                            jnp.broadcast_to(entry, (16,)),
                            mask=one_lane,
                        )

                    @pl.when(slot >= _KCAP)
                    def _():
                        cnt_s[of_slot] = jnp.int32(1)
            return x

        lax.fori_loop(0, n_grp, bk_body, 0)
        overflow = cnt_s[of_slot]

        def drain_one(_, x):
            pltpu.make_async_copy(
                rowb_v.at[0], packed_hbm.at[pl.ds(0, D)], sem_r
            ).wait()
            return x

        def extract(buf, entry, rb):
            pos = entry // 1024
            off = jnp.broadcast_to(entry % 1024, (16,))
            bufi = jnp.broadcast_to(buf, (16,))
            for q in range(D // 16):
                col = plsc.load_gather(
                    chunk_v, [bufi, iota + q * 16, off]
                )
                rowb_v[rb, pl.ds(q * 16, 16)] = col
            pltpu.async_copy(
                rowb_v.at[rb], packed_hbm.at[pl.ds(pos * D, D)], sem_r
            )

        def ck_body(c, n_prev):
            buf = lax.rem(c, 2)
            @pl.when(buf == 0)
            def _():
                pltpu.make_async_copy(
                    tT_hbm.at[:, pl.ds(lo + c * _CH, _CH)],
                    chunk_v.at[0],
                    sem_c0,
                ).wait()

            @pl.when(buf == 1)
            def _():
                pltpu.make_async_copy(
                    tT_hbm.at[:, pl.ds(lo + c * _CH, _CH)],
                    chunk_v.at[1],
                    sem_c1,
                ).wait()

            def fast(x):
                nb = jnp.minimum(cnt_s[c], _KCAP)
                sbase = buf * _KCAP

                def hit_body(t, y):
                    entry = bk_v[pl.ds(c * _KCAP + t, 16)][0]
                    extract(buf, entry, sbase + t)
                    return y

                lax.fori_loop(0, nb, hit_body, 0)
                return nb

            def slow(x):
                off = lo + c * _CH
                sbase = buf * _KCAP

                def grp_body(hg, n_pg):
                    lax.fori_loop(0, n_pg, drain_one, 0)
                    hv = hitval_v[pl.ds(hg * 16, 16)]
                    hp = hitpos_v[pl.ds(hg * 16, 16)]
                    mi = ((hv >= off) & (hv < off + _CH)).astype(jnp.int32)
                    for e in range(16):
                        @pl.when(mi[e] != 0)
                        def _():
                            entry = hp[e] * 1024 + (hv[e] - off)
                            extract(buf, entry, sbase + e)
                    return plsc.all_reduce_population_count(mi != 0)[0]

                return lax.fori_loop(0, n_grp, grp_body, jnp.int32(0))

            nb = lax.cond(overflow == 0, fast, slow, 0)

            @pl.when((c + 2 < n_ck) & (buf == 0))
            def _():
                pltpu.make_async_copy(
                    tT_hbm.at[:, pl.ds(lo + (c + 2) * _CH, _CH)],
                    chunk_v.at[0],
                    sem_c0,
                ).start()

            @pl.when((c + 2 < n_ck) & (buf == 1))
            def _():
                pltpu.make_async_copy(
                    tT_hbm.at[:, pl.ds(lo + (c + 2) * _CH, _CH)],
                    chunk_v.at[1],
                    sem_c1,
                ).start()

            return nb

        def ck_outer(c, carry):
            n1, n2 = carry
            lax.fori_loop(0, n2, drain_one, 0)
            n0 = ck_body(c, jnp.int32(0))
            return (n0, n1)

        n1, n2 = lax.fori_loop(
            0, n_ck, ck_outer, (jnp.int32(0), jnp.int32(0))
        )
        lax.fori_loop(0, n1 + n2, drain_one, 0)

        @pl.when(is_last)
        def _():
            def tail_body(hg, carry):
                hv = hitval_v[pl.ds(hg * 16, 16)]
                hp = hitpos_v[pl.ds(hg * 16, 16)]
                m3i = (hv >= aligned_v).astype(jnp.int32)
                for e in range(16):
                    @pl.when(m3i[e] != 0)
                    def _():
                        pltpu.make_async_copy(
                            tail_hbm.at[pl.ds((hv[e] - aligned_v) * D, D)],
                            rowb_v.at[0],
                            sem_r,
                        ).start()
                        pltpu.make_async_copy(
                            tail_hbm.at[pl.ds(0, D)], rowb_v.at[0], sem_r
                        ).wait()
                        pltpu.async_copy(
                            rowb_v.at[0],
                            packed_hbm.at[pl.ds(hp[e] * D, D)],
                            sem_r,
                        ).wait()
                return carry
            lax.fori_loop(0, n_grp, tail_body, 0)

    return k


def _phase2_kernel(B, D, NC, NS):
    NW = NC * NS
    bw = B // NW
    mesh = plsc.VectorSubcoreMesh(core_axis_name="c", subcore_axis_name="s")

    oddw = bw + 15

    @functools.partial(
        pl.kernel,
        mesh=mesh,
        compiler_params=pltpu.CompilerParams(needs_layout_passes=False),
        out_type=jax.ShapeDtypeStruct((D, B), jnp.float32),
        scratch_types=[
            pltpu.VMEM((bw * D,), jnp.float32),
            pltpu.VMEM((D, oddw), jnp.float32),
        ],
    )
    def k(packed_hbm, outT_hbm, pk_v, out_v):
        wid = lax.axis_index("s") * NC + lax.axis_index("c")
        base = wid * bw
        pltpu.sync_copy(packed_hbm.at[pl.ds(base * D, bw * D)], pk_v)
        iota = lax.iota(jnp.int32, 16)
        rows_q = [iota + q * 16 for q in range(D // 16)]

        def eg_body(eg, carry):
            for es in range(16):
                e = eg * 16 + es
                col = jnp.broadcast_to(e, (16,))
                rowbase = e * D
                for q in range(D // 16):
                    vals = pk_v[pl.ds(rowbase + q * 16, 16)]
                    plsc.store_scatter(out_v, [rows_q[q], col], vals)
            return carry

        lax.fori_loop(0, bw // 16, eg_body, 0)
        pltpu.sync_copy(
            out_v.at[:, pl.ds(0, bw)], outT_hbm.at[:, pl.ds(base, bw)]
        )

    return k


def kernel(indices, codes_weight):
    if indices.ndim > 1:
        indices = jnp.squeeze(indices, axis=-1)
    B = indices.shape[0]
    V, D = codes_weight.shape
    info = plsc.get_sparse_core_info()
    NC, NS = info.num_cores, info.num_subcores
    idx = indices.astype(jnp.int32)
    tT = codes_weight.T
    aligned_v = (V // _CH) * _CH
    tail_flat = codes_weight[aligned_v:, :].reshape(-1)
    packed = _phase1_kernel(B, V, D, NC, NS)(idx, tT, tail_flat)
    return packed.reshape(B, D)


# speedup vs baseline: 7.8282x; 1.1061x over previous
import functools

import jax
import jax.numpy as jnp
from jax import lax
from jax.experimental import pallas as pl
from jax.experimental.pallas import tpu as pltpu
from jax.experimental.pallas import tpu_sc as plsc

_CH = 256
_NBUF = 4
_KCAP = 32
_NROWB = 64


def _phase1_kernel(B, V, D, NC, NS):
    NW = NC * NS
    n_ck_total = V // _CH
    aligned_v = n_ck_total * _CH
    base_ck = n_ck_total // NW
    n_extra = n_ck_total - base_ck * NW
    max_ck = base_ck + 1
    mesh = plsc.VectorSubcoreMesh(core_axis_name="c", subcore_axis_name="s")

    @functools.partial(
        pl.kernel,
        mesh=mesh,
        compiler_params=pltpu.CompilerParams(needs_layout_passes=False),
        out_type=jax.ShapeDtypeStruct((B * D,), jnp.float32),
        scratch_types=[
            pltpu.VMEM((B,), jnp.int32),
            pltpu.VMEM((B + 16,), jnp.int32),
            pltpu.VMEM((B + 16,), jnp.int32),
            pltpu.VMEM((max_ck * _KCAP + 16,), jnp.int32),
            pltpu.VMEM((_NBUF, D, _CH), jnp.float32),
            pltpu.VMEM((_NROWB, D), jnp.float32),
            pltpu.SMEM((max_ck + 8,), jnp.int32),
            pltpu.SemaphoreType.DMA,
            pltpu.SemaphoreType.DMA,
            pltpu.SemaphoreType.DMA,
            pltpu.SemaphoreType.DMA,
            pltpu.SemaphoreType.DMA,
        ],
    )
    def k(idx_hbm, tT_hbm, tail_hbm, packed_hbm, idx_v, hitpos_v, hitval_v,
          bk_v, chunk_v, rowb_v, cnt_s, sem_r, sem_c0, sem_c1, sem_c2,
          sem_c3):
        wid = lax.axis_index("s") * NC + lax.axis_index("c")
        n_ck = base_ck + jnp.where(wid < n_extra, 1, 0)
        start_ck = wid * base_ck + jnp.minimum(wid, n_extra)
        lo = start_ck * _CH
        hi = lo + n_ck * _CH
        is_last = wid == NW - 1
        sel_hi = jnp.where(is_last, V, hi)
        of_slot = max_ck + 1

        _sems = [sem_c0, sem_c1, sem_c2, sem_c3]
        for b in range(_NBUF):
            pltpu.make_async_copy(
                tT_hbm.at[:, pl.ds(lo + b * _CH, _CH)], chunk_v.at[b],
                _sems[b],
            ).start()

        pltpu.sync_copy(idx_hbm, idx_v)
        iota = lax.iota(jnp.int32, 16)

        def sel_body(g, cnt):
            v = idx_v[pl.ds(g * 16, 16)]
            m = (v >= lo) & (v < sel_hi)
            pos = g * 16 + iota
            plsc.store_compressed(hitpos_v.at[pl.ds(cnt, 16)], pos, mask=m)
            plsc.store_compressed(hitval_v.at[pl.ds(cnt, 16)], v, mask=m)
            return cnt + plsc.all_reduce_population_count(m)[0]

        cnt = lax.fori_loop(0, B // 16, sel_body, jnp.int32(0))
        hitval_v[pl.ds(cnt, 16)] = jnp.full((16,), -1, jnp.int32)
        n_grp = (cnt + 15) // 16

        def zero_body(c, x):
            cnt_s[c] = jnp.int32(0)
            return x
        lax.fori_loop(0, max_ck + 8, zero_body, 0)

        one_lane = iota == 0

        def bk_body(hg, x):
            hv = hitval_v[pl.ds(hg * 16, 16)]
            hp = hitpos_v[pl.ds(hg * 16, 16)]
            mi = ((hv >= lo) & (hv < hi)).astype(jnp.int32)
            for e in range(16):
                @pl.when(mi[e] != 0)
                def _():
                    rel = hv[e] - lo
                    c = rel // _CH
                    slot = cnt_s[c]
                    cnt_s[c] = slot + 1

                    @pl.when(slot < _KCAP)
                    def _():
                        entry = (hp[e] * 1024) + (rel % _CH)
                        plsc.store_compressed(
                            bk_v.at[pl.ds(c * _KCAP + slot, 16)],
                            jnp.broadcast_to(entry, (16,)),
                            mask=one_lane,
                        )

                    @pl.when(slot >= _KCAP)
                    def _():
                        cnt_s[of_slot] = jnp.int32(1)
            return x

        lax.fori_loop(0, n_grp, bk_body, 0)
        overflow = cnt_s[of_slot]

        def drain_one(_, x):
            pltpu.make_async_copy(
                rowb_v.at[0], packed_hbm.at[pl.ds(0, D)], sem_r
            ).wait()
            return x

        def extract(buf, entry, rb):
            pos = entry // 1024
            off = jnp.broadcast_to(entry % 1024, (16,))
            bufi = jnp.broadcast_to(buf, (16,))
            for q in range(D // 16):
                col = plsc.load_gather(
                    chunk_v, [bufi, iota + q * 16, off]
                )
                rowb_v[rb, pl.ds(q * 16, 16)] = col
            pltpu.async_copy(
                rowb_v.at[rb], packed_hbm.at[pl.ds(pos * D, D)], sem_r
            )

        def ck_body(c, n_prev):
            buf = lax.rem(c, _NBUF)
            for b in range(_NBUF):
                @pl.when(buf == b)
                def _(b=b):
                    pltpu.make_async_copy(
                        tT_hbm.at[:, pl.ds(lo + c * _CH, _CH)],
                        chunk_v.at[b],
                        _sems[b],
                    ).wait()

            def fast(x):
                nb = jnp.minimum(cnt_s[c], _KCAP)
                sbase = lax.rem(c, 2) * _KCAP

                def hit_body(t, y):
                    entry = bk_v[pl.ds(c * _KCAP + t, 16)][0]
                    extract(buf, entry, sbase + t)
                    return y

                lax.fori_loop(0, nb, hit_body, 0)
                return nb

            def slow(x):
                off = lo + c * _CH
                sbase = lax.rem(c, 2) * _KCAP

                def grp_body(hg, n_pg):
                    lax.fori_loop(0, n_pg, drain_one, 0)
                    hv = hitval_v[pl.ds(hg * 16, 16)]
                    hp = hitpos_v[pl.ds(hg * 16, 16)]
                    mi = ((hv >= off) & (hv < off + _CH)).astype(jnp.int32)
                    for e in range(16):
                        @pl.when(mi[e] != 0)
                        def _():
                            entry = hp[e] * 1024 + (hv[e] - off)
                            extract(buf, entry, sbase + e)
                    return plsc.all_reduce_population_count(mi != 0)[0]

                return lax.fori_loop(0, n_grp, grp_body, jnp.int32(0))

            nb = lax.cond(overflow == 0, fast, slow, 0)

            for b in range(_NBUF):
                @pl.when((c + _NBUF < n_ck) & (buf == b))
                def _(b=b):
                    pltpu.make_async_copy(
                        tT_hbm.at[:, pl.ds(lo + (c + _NBUF) * _CH, _CH)],
                        chunk_v.at[b],
                        _sems[b],
                    ).start()

            return nb

        def ck_outer(c, carry):
            n1, n2 = carry
            lax.fori_loop(0, n2, drain_one, 0)
            n0 = ck_body(c, jnp.int32(0))
            return (n0, n1)

        n1, n2 = lax.fori_loop(
            0, n_ck, ck_outer, (jnp.int32(0), jnp.int32(0))
        )
        lax.fori_loop(0, n1 + n2, drain_one, 0)

        @pl.when(is_last)
        def _():
            def tail_body(hg, carry):
                hv = hitval_v[pl.ds(hg * 16, 16)]
                hp = hitpos_v[pl.ds(hg * 16, 16)]
                m3i = (hv >= aligned_v).astype(jnp.int32)
                for e in range(16):
                    @pl.when(m3i[e] != 0)
                    def _():
                        pltpu.make_async_copy(
                            tail_hbm.at[pl.ds((hv[e] - aligned_v) * D, D)],
                            rowb_v.at[0],
                            sem_r,
                        ).start()
                        pltpu.make_async_copy(
                            tail_hbm.at[pl.ds(0, D)], rowb_v.at[0], sem_r
                        ).wait()
                        pltpu.async_copy(
                            rowb_v.at[0],
                            packed_hbm.at[pl.ds(hp[e] * D, D)],
                            sem_r,
                        ).wait()
                return carry
            lax.fori_loop(0, n_grp, tail_body, 0)

    return k


def _phase2_kernel(B, D, NC, NS):
    NW = NC * NS
    bw = B // NW
    mesh = plsc.VectorSubcoreMesh(core_axis_name="c", subcore_axis_name="s")

    oddw = bw + 15

    @functools.partial(
        pl.kernel,
        mesh=mesh,
        compiler_params=pltpu.CompilerParams(needs_layout_passes=False),
        out_type=jax.ShapeDtypeStruct((D, B), jnp.float32),
        scratch_types=[
            pltpu.VMEM((bw * D,), jnp.float32),
            pltpu.VMEM((D, oddw), jnp.float32),
        ],
    )
    def k(packed_hbm, outT_hbm, pk_v, out_v):
        wid = lax.axis_index("s") * NC + lax.axis_index("c")
        base = wid * bw
        pltpu.sync_copy(packed_hbm.at[pl.ds(base * D, bw * D)], pk_v)
        iota = lax.iota(jnp.int32, 16)
        rows_q = [iota + q * 16 for q in range(D // 16)]

        def eg_body(eg, carry):
            for es in range(16):
                e = eg * 16 + es
                col = jnp.broadcast_to(e, (16,))
                rowbase = e * D
                for q in range(D // 16):
                    vals = pk_v[pl.ds(rowbase + q * 16, 16)]
                    plsc.store_scatter(out_v, [rows_q[q], col], vals)
            return carry

        lax.fori_loop(0, bw // 16, eg_body, 0)
        pltpu.sync_copy(
            out_v.at[:, pl.ds(0, bw)], outT_hbm.at[:, pl.ds(base, bw)]
        )

    return k


def kernel(indices, codes_weight):
    if indices.ndim > 1:
        indices = jnp.squeeze(indices, axis=-1)
    B = indices.shape[0]
    V, D = codes_weight.shape
    info = plsc.get_sparse_core_info()
    NC, NS = info.num_cores, info.num_subcores
    idx = indices.astype(jnp.int32)
    tT = codes_weight.T
    aligned_v = (V // _CH) * _CH
    tail_flat = codes_weight[aligned_v:, :].reshape(-1)
    packed = _phase1_kernel(B, V, D, NC, NS)(idx, tT, tail_flat)
    return packed.reshape(B, D)


# speedup vs baseline: 7.8675x; 1.0050x over previous
import functools

import jax
import jax.numpy as jnp
from jax import lax
from jax.experimental import pallas as pl
from jax.experimental.pallas import tpu as pltpu
from jax.experimental.pallas import tpu_sc as plsc

_CH = 256
_NBUF = 4
_KCAP = 32
_NROWB = 64


def _phase1_kernel(B, V, D, NC, NS):
    NW = NC * NS
    n_ck_total = V // _CH
    aligned_v = n_ck_total * _CH
    base_ck = n_ck_total // NW
    n_extra = n_ck_total - base_ck * NW
    max_ck = base_ck + 1
    mesh = plsc.VectorSubcoreMesh(core_axis_name="c", subcore_axis_name="s")

    @functools.partial(
        pl.kernel,
        mesh=mesh,
        compiler_params=pltpu.CompilerParams(needs_layout_passes=False),
        out_type=jax.ShapeDtypeStruct((B * D,), jnp.float32),
        scratch_types=[
            pltpu.VMEM((B,), jnp.int32),
            pltpu.VMEM((B + 16,), jnp.int32),
            pltpu.VMEM((B + 16,), jnp.int32),
            pltpu.VMEM((max_ck * _KCAP + 16,), jnp.int32),
            pltpu.VMEM((_NBUF, D, _CH), jnp.float32),
            pltpu.VMEM((_NROWB, D), jnp.float32),
            pltpu.SMEM((max_ck + 8,), jnp.int32),
            pltpu.SemaphoreType.DMA,
            pltpu.SemaphoreType.DMA,
            pltpu.SemaphoreType.DMA,
            pltpu.SemaphoreType.DMA,
            pltpu.SemaphoreType.DMA,
        ],
    )
    def k(idx_hbm, tT_hbm, tail_hbm, packed_hbm, idx_v, hitpos_v, hitval_v,
          bk_v, chunk_v, rowb_v, cnt_s, sem_r, sem_c0, sem_c1, sem_c2,
          sem_c3):
        wid = lax.axis_index("s") * NC + lax.axis_index("c")
        n_ck = base_ck + jnp.where(wid < n_extra, 1, 0)
        start_ck = wid * base_ck + jnp.minimum(wid, n_extra)
        lo = start_ck * _CH
        hi = lo + n_ck * _CH
        is_last = wid == NW - 1
        sel_hi = jnp.where(is_last, V, hi)
        of_slot = max_ck + 1

        _sems = [sem_c0, sem_c1, sem_c2, sem_c3]
        for b in range(_NBUF):
            pltpu.make_async_copy(
                tT_hbm.at[:, pl.ds(lo + b * _CH, _CH)], chunk_v.at[b],
                _sems[b],
            ).start()

        pltpu.sync_copy(idx_hbm, idx_v)
        iota = lax.iota(jnp.int32, 16)

        def sel_body(g, cnt):
            v = idx_v[pl.ds(g * 16, 16)]
            m = (v >= lo) & (v < sel_hi)
            pos = g * 16 + iota
            plsc.store_compressed(hitpos_v.at[pl.ds(cnt, 16)], pos, mask=m)
            plsc.store_compressed(hitval_v.at[pl.ds(cnt, 16)], v, mask=m)
            return cnt + plsc.all_reduce_population_count(m)[0]

        cnt = lax.fori_loop(0, B // 16, sel_body, jnp.int32(0))
        hitval_v[pl.ds(cnt, 16)] = jnp.full((16,), -1, jnp.int32)
        n_grp = (cnt + 15) // 16

        def zero_body(c, x):
            cnt_s[c] = jnp.int32(0)
            return x
        lax.fori_loop(0, max_ck + 8, zero_body, 0)

        one_lane = iota == 0

        def bk_body(hg, x):
            hv = hitval_v[pl.ds(hg * 16, 16)]
            hp = hitpos_v[pl.ds(hg * 16, 16)]
            mi = ((hv >= lo) & (hv < hi)).astype(jnp.int32)
            for e in range(16):
                @pl.when(mi[e] != 0)
                def _():
                    rel = hv[e] - lo
                    c = rel // _CH
                    slot = cnt_s[c]
                    cnt_s[c] = slot + 1

                    @pl.when(slot < _KCAP)
                    def _():
                        entry = (hp[e] * 1024) + (rel % _CH)
                        plsc.store_compressed(
                            bk_v.at[pl.ds(c * _KCAP + slot, 16)],
                            jnp.broadcast_to(entry, (16,)),
                            mask=one_lane,
                        )

                    @pl.when(slot >= _KCAP)
                    def _():
                        cnt_s[of_slot] = jnp.int32(1)
            return x

        lax.fori_loop(0, n_grp, bk_body, 0)
        overflow = cnt_s[of_slot]

        def drain_one(_, x):
            pltpu.make_async_copy(
                rowb_v.at[0], packed_hbm.at[pl.ds(0, D)], sem_r
            ).wait()
            return x

        def extract(buf, entry, rb):
            pos = entry // 1024
            off = jnp.broadcast_to(entry % 1024, (16,))
            bufi = jnp.broadcast_to(buf, (16,))
            for q in range(D // 16):
                col = plsc.load_gather(
                    chunk_v, [bufi, iota + q * 16, off]
                )
                rowb_v[rb, pl.ds(q * 16, 16)] = col
            pltpu.async_copy(
                rowb_v.at[rb], packed_hbm.at[pl.ds(pos * D, D)], sem_r
            )

        def ck_body(c, n_prev):
            buf = lax.rem(c, _NBUF)
            for b in range(_NBUF):
                @pl.when(buf == b)
                def _(b=b):
                    pltpu.make_async_copy(
                        tT_hbm.at[:, pl.ds(lo + c * _CH, _CH)],
                        chunk_v.at[b],
                        _sems[b],
                    ).wait()

            def fast(x):
                nb = jnp.minimum(cnt_s[c], _KCAP)
                sbase = 0

                def hit_body(t, y):
                    entry = bk_v[pl.ds(c * _KCAP + t, 16)][0]
                    extract(buf, entry, sbase + t)
                    return y

                lax.fori_loop(0, nb, hit_body, 0)
                return nb

            def slow(x):
                off = lo + c * _CH
                sbase = 0

                def grp_body(hg, n_pg):
                    lax.fori_loop(0, n_pg, drain_one, 0)
                    hv = hitval_v[pl.ds(hg * 16, 16)]
                    hp = hitpos_v[pl.ds(hg * 16, 16)]
                    mi = ((hv >= off) & (hv < off + _CH)).astype(jnp.int32)
                    for e in range(16):
                        @pl.when(mi[e] != 0)
                        def _():
                            entry = hp[e] * 1024 + (hv[e] - off)
                            extract(buf, entry, sbase + e)
                    return plsc.all_reduce_population_count(mi != 0)[0]

                return lax.fori_loop(0, n_grp, grp_body, jnp.int32(0))

            nb = lax.cond(overflow == 0, fast, slow, 0)

            for b in range(_NBUF):
                @pl.when((c + _NBUF < n_ck) & (buf == b))
                def _(b=b):
                    pltpu.make_async_copy(
                        tT_hbm.at[:, pl.ds(lo + (c + _NBUF) * _CH, _CH)],
                        chunk_v.at[b],
                        _sems[b],
                    ).start()

            return nb

        def ck_outer(c, n_pending):
            lax.fori_loop(0, n_pending, drain_one, 0)
            return ck_body(c, jnp.int32(0))

        n_pending = lax.fori_loop(0, n_ck, ck_outer, jnp.int32(0))
        lax.fori_loop(0, n_pending, drain_one, 0)

        @pl.when(is_last)
        def _():
            def tail_body(hg, carry):
                hv = hitval_v[pl.ds(hg * 16, 16)]
                hp = hitpos_v[pl.ds(hg * 16, 16)]
                m3i = (hv >= aligned_v).astype(jnp.int32)
                for e in range(16):
                    @pl.when(m3i[e] != 0)
                    def _():
                        pltpu.make_async_copy(
                            tail_hbm.at[pl.ds((hv[e] - aligned_v) * D, D)],
                            rowb_v.at[0],
                            sem_r,
                        ).start()
                        pltpu.make_async_copy(
                            tail_hbm.at[pl.ds(0, D)], rowb_v.at[0], sem_r
                        ).wait()
                        pltpu.async_copy(
                            rowb_v.at[0],
                            packed_hbm.at[pl.ds(hp[e] * D, D)],
                            sem_r,
                        ).wait()
                return carry
            lax.fori_loop(0, n_grp, tail_body, 0)

    return k


def _phase2_kernel(B, D, NC, NS):
    NW = NC * NS
    bw = B // NW
    mesh = plsc.VectorSubcoreMesh(core_axis_name="c", subcore_axis_name="s")

    oddw = bw + 15

    @functools.partial(
        pl.kernel,
        mesh=mesh,
        compiler_params=pltpu.CompilerParams(needs_layout_passes=False),
        out_type=jax.ShapeDtypeStruct((D, B), jnp.float32),
        scratch_types=[
            pltpu.VMEM((bw * D,), jnp.float32),
            pltpu.VMEM((D, oddw), jnp.float32),
        ],
    )
    def k(packed_hbm, outT_hbm, pk_v, out_v):
        wid = lax.axis_index("s") * NC + lax.axis_index("c")
        base = wid * bw
        pltpu.sync_copy(packed_hbm.at[pl.ds(base * D, bw * D)], pk_v)
        iota = lax.iota(jnp.int32, 16)
        rows_q = [iota + q * 16 for q in range(D // 16)]

        def eg_body(eg, carry):
            for es in range(16):
                e = eg * 16 + es
                col = jnp.broadcast_to(e, (16,))
                rowbase = e * D
                for q in range(D // 16):
                    vals = pk_v[pl.ds(rowbase + q * 16, 16)]
                    plsc.store_scatter(out_v, [rows_q[q], col], vals)
            return carry

        lax.fori_loop(0, bw // 16, eg_body, 0)
        pltpu.sync_copy(
            out_v.at[:, pl.ds(0, bw)], outT_hbm.at[:, pl.ds(base, bw)]
        )

    return k


def kernel(indices, codes_weight):
    if indices.ndim > 1:
        indices = jnp.squeeze(indices, axis=-1)
    B = indices.shape[0]
    V, D = codes_weight.shape
    info = plsc.get_sparse_core_info()
    NC, NS = info.num_cores, info.num_subcores
    idx = indices.astype(jnp.int32)
    tT = codes_weight.T
    aligned_v = (V // _CH) * _CH
    tail_flat = codes_weight[aligned_v:, :].reshape(-1)
    packed = _phase1_kernel(B, V, D, NC, NS)(idx, tT, tail_flat)
    return packed.reshape(B, D)


# speedup vs baseline: 8.3164x; 1.0571x over previous
import functools

import jax
import jax.numpy as jnp
from jax import lax
from jax.experimental import pallas as pl
from jax.experimental.pallas import tpu as pltpu
from jax.experimental.pallas import tpu_sc as plsc

_CH = 128
_NBUF = 8
_KCAP = 32
_NROWB = 64


def _phase1_kernel(B, V, D, NC, NS):
    NW = NC * NS
    n_ck_total = V // _CH
    aligned_v = n_ck_total * _CH
    base_ck = n_ck_total // NW
    n_extra = n_ck_total - base_ck * NW
    max_ck = base_ck + 1
    mesh = plsc.VectorSubcoreMesh(core_axis_name="c", subcore_axis_name="s")

    @functools.partial(
        pl.kernel,
        mesh=mesh,
        compiler_params=pltpu.CompilerParams(needs_layout_passes=False),
        out_type=jax.ShapeDtypeStruct((B * D,), jnp.float32),
        scratch_types=[
            pltpu.VMEM((B,), jnp.int32),
            pltpu.VMEM((B + 16,), jnp.int32),
            pltpu.VMEM((B + 16,), jnp.int32),
            pltpu.VMEM((max_ck * _KCAP + 16,), jnp.int32),
            pltpu.VMEM((_NBUF, D, _CH), jnp.float32),
            pltpu.VMEM((_NROWB, D), jnp.float32),
            pltpu.SMEM((max_ck + 8,), jnp.int32),
            pltpu.SemaphoreType.DMA,
            pltpu.SemaphoreType.DMA,
            pltpu.SemaphoreType.DMA,
            pltpu.SemaphoreType.DMA,
            pltpu.SemaphoreType.DMA,
            pltpu.SemaphoreType.DMA,
            pltpu.SemaphoreType.DMA,
            pltpu.SemaphoreType.DMA,
            pltpu.SemaphoreType.DMA,
        ],
    )
    def k(idx_hbm, tT_hbm, tail_hbm, packed_hbm, idx_v, hitpos_v, hitval_v,
          bk_v, chunk_v, rowb_v, cnt_s, sem_r, sem_c0, sem_c1, sem_c2,
          sem_c3, sem_c4, sem_c5, sem_c6, sem_c7):
        wid = lax.axis_index("s") * NC + lax.axis_index("c")
        n_ck = base_ck + jnp.where(wid < n_extra, 1, 0)
        start_ck = wid * base_ck + jnp.minimum(wid, n_extra)
        lo = start_ck * _CH
        hi = lo + n_ck * _CH
        is_last = wid == NW - 1
        sel_hi = jnp.where(is_last, V, hi)
        of_slot = max_ck + 1

        _sems = [sem_c0, sem_c1, sem_c2, sem_c3, sem_c4, sem_c5, sem_c6, sem_c7]
        for b in range(_NBUF):
            pltpu.make_async_copy(
                tT_hbm.at[:, pl.ds(lo + b * _CH, _CH)], chunk_v.at[b],
                _sems[b],
            ).start()

        pltpu.sync_copy(idx_hbm, idx_v)
        iota = lax.iota(jnp.int32, 16)

        def sel_body(g, cnt):
            v = idx_v[pl.ds(g * 16, 16)]
            m = (v >= lo) & (v < sel_hi)
            pos = g * 16 + iota
            plsc.store_compressed(hitpos_v.at[pl.ds(cnt, 16)], pos, mask=m)
            plsc.store_compressed(hitval_v.at[pl.ds(cnt, 16)], v, mask=m)
            return cnt + plsc.all_reduce_population_count(m)[0]

        cnt = lax.fori_loop(0, B // 16, sel_body, jnp.int32(0))
        hitval_v[pl.ds(cnt, 16)] = jnp.full((16,), -1, jnp.int32)
        n_grp = (cnt + 15) // 16

        def zero_body(c, x):
            cnt_s[c] = jnp.int32(0)
            return x
        lax.fori_loop(0, max_ck + 8, zero_body, 0)

        one_lane = iota == 0

        def bk_body(hg, x):
            hv = hitval_v[pl.ds(hg * 16, 16)]
            hp = hitpos_v[pl.ds(hg * 16, 16)]
            mi = ((hv >= lo) & (hv < hi)).astype(jnp.int32)
            for e in range(16):
                @pl.when(mi[e] != 0)
                def _():
                    rel = hv[e] - lo
                    c = rel // _CH
                    slot = cnt_s[c]
                    cnt_s[c] = slot + 1

                    @pl.when(slot < _KCAP)
                    def _():
                        entry = (hp[e] * 1024) + (rel % _CH)
                        plsc.store_compressed(
                            bk_v.at[pl.ds(c * _KCAP + slot, 16)],
                            jnp.broadcast_to(entry, (16,)),
                            mask=one_lane,
                        )

                    @pl.when(slot >= _KCAP)
                    def _():
                        cnt_s[of_slot] = jnp.int32(1)
            return x

        lax.fori_loop(0, n_grp, bk_body, 0)
        overflow = cnt_s[of_slot]

        def drain_one(_, x):
            pltpu.make_async_copy(
                rowb_v.at[0], packed_hbm.at[pl.ds(0, D)], sem_r
            ).wait()
            return x

        def extract(buf, entry, rb):
            pos = entry // 1024
            off = jnp.broadcast_to(entry % 1024, (16,))
            bufi = jnp.broadcast_to(buf, (16,))
            for q in range(D // 16):
                col = plsc.load_gather(
                    chunk_v, [bufi, iota + q * 16, off]
                )
                rowb_v[rb, pl.ds(q * 16, 16)] = col
            pltpu.async_copy(
                rowb_v.at[rb], packed_hbm.at[pl.ds(pos * D, D)], sem_r
            )

        def ck_body(c, n_prev):
            buf = lax.rem(c, _NBUF)
            for b in range(_NBUF):
                @pl.when(buf == b)
                def _(b=b):
                    pltpu.make_async_copy(
                        tT_hbm.at[:, pl.ds(lo + c * _CH, _CH)],
                        chunk_v.at[b],
                        _sems[b],
                    ).wait()

            def fast(x):
                nb = jnp.minimum(cnt_s[c], _KCAP)
                sbase = 0

                def hit_body(t, y):
                    entry = bk_v[pl.ds(c * _KCAP + t, 16)][0]
                    extract(buf, entry, sbase + t)
                    return y

                lax.fori_loop(0, nb, hit_body, 0)
                return nb

            def slow(x):
                off = lo + c * _CH
                sbase = 0

                def grp_body(hg, n_pg):
                    lax.fori_loop(0, n_pg, drain_one, 0)
                    hv = hitval_v[pl.ds(hg * 16, 16)]
                    hp = hitpos_v[pl.ds(hg * 16, 16)]
                    mi = ((hv >= off) & (hv < off + _CH)).astype(jnp.int32)
                    for e in range(16):
                        @pl.when(mi[e] != 0)
                        def _():
                            entry = hp[e] * 1024 + (hv[e] - off)
                            extract(buf, entry, sbase + e)
                    return plsc.all_reduce_population_count(mi != 0)[0]

                return lax.fori_loop(0, n_grp, grp_body, jnp.int32(0))

            nb = lax.cond(overflow == 0, fast, slow, 0)

            for b in range(_NBUF):
                @pl.when((c + _NBUF < n_ck) & (buf == b))
                def _(b=b):
                    pltpu.make_async_copy(
                        tT_hbm.at[:, pl.ds(lo + (c + _NBUF) * _CH, _CH)],
                        chunk_v.at[b],
                        _sems[b],
                    ).start()

            return nb

        def ck_outer(c, n_pending):
            lax.fori_loop(0, n_pending, drain_one, 0)
            return ck_body(c, jnp.int32(0))

        n_pending = lax.fori_loop(0, n_ck, ck_outer, jnp.int32(0))
        lax.fori_loop(0, n_pending, drain_one, 0)

        @pl.when(is_last)
        def _():
            def tail_body(hg, carry):
                hv = hitval_v[pl.ds(hg * 16, 16)]
                hp = hitpos_v[pl.ds(hg * 16, 16)]
                m3i = (hv >= aligned_v).astype(jnp.int32)
                for e in range(16):
                    @pl.when(m3i[e] != 0)
                    def _():
                        pltpu.make_async_copy(
                            tail_hbm.at[pl.ds((hv[e] - aligned_v) * D, D)],
                            rowb_v.at[0],
                            sem_r,
                        ).start()
                        pltpu.make_async_copy(
                            tail_hbm.at[pl.ds(0, D)], rowb_v.at[0], sem_r
                        ).wait()
                        pltpu.async_copy(
                            rowb_v.at[0],
                            packed_hbm.at[pl.ds(hp[e] * D, D)],
                            sem_r,
                        ).wait()
                return carry
            lax.fori_loop(0, n_grp, tail_body, 0)

    return k


def _phase2_kernel(B, D, NC, NS):
    NW = NC * NS
    bw = B // NW
    mesh = plsc.VectorSubcoreMesh(core_axis_name="c", subcore_axis_name="s")

    oddw = bw + 15

    @functools.partial(
        pl.kernel,
        mesh=mesh,
        compiler_params=pltpu.CompilerParams(needs_layout_passes=False),
        out_type=jax.ShapeDtypeStruct((D, B), jnp.float32),
        scratch_types=[
            pltpu.VMEM((bw * D,), jnp.float32),
            pltpu.VMEM((D, oddw), jnp.float32),
        ],
    )
    def k(packed_hbm, outT_hbm, pk_v, out_v):
        wid = lax.axis_index("s") * NC + lax.axis_index("c")
        base = wid * bw
        pltpu.sync_copy(packed_hbm.at[pl.ds(base * D, bw * D)], pk_v)
        iota = lax.iota(jnp.int32, 16)
        rows_q = [iota + q * 16 for q in range(D // 16)]

        def eg_body(eg, carry):
            for es in range(16):
                e = eg * 16 + es
                col = jnp.broadcast_to(e, (16,))
                rowbase = e * D
                for q in range(D // 16):
                    vals = pk_v[pl.ds(rowbase + q * 16, 16)]
                    plsc.store_scatter(out_v, [rows_q[q], col], vals)
            return carry

        lax.fori_loop(0, bw // 16, eg_body, 0)
        pltpu.sync_copy(
            out_v.at[:, pl.ds(0, bw)], outT_hbm.at[:, pl.ds(base, bw)]
        )

    return k


def kernel(indices, codes_weight):
    if indices.ndim > 1:
        indices = jnp.squeeze(indices, axis=-1)
    B = indices.shape[0]
    V, D = codes_weight.shape
    info = plsc.get_sparse_core_info()
    NC, NS = info.num_cores, info.num_subcores
    idx = indices.astype(jnp.int32)
    tT = codes_weight.T
    aligned_v = (V // _CH) * _CH
    tail_flat = codes_weight[aligned_v:, :].reshape(-1)
    packed = _phase1_kernel(B, V, D, NC, NS)(idx, tT, tail_flat)
    return packed.reshape(B, D)


# speedup vs baseline: 8.3213x; 1.0006x over previous
import functools

import jax
import jax.numpy as jnp
from jax import lax
from jax.experimental import pallas as pl
from jax.experimental.pallas import tpu as pltpu
from jax.experimental.pallas import tpu_sc as plsc

_CH = 128
_NBUF = 8
_KCAP = 32
_NROWB = 64


def _phase1_kernel(B, V, D, NC, NS):
    NW = NC * NS
    n_ck_total = V // _CH
    aligned_v = n_ck_total * _CH
    base_ck = n_ck_total // NW
    n_extra = n_ck_total - base_ck * NW
    max_ck = base_ck + 1
    mesh = plsc.VectorSubcoreMesh(core_axis_name="c", subcore_axis_name="s")

    @functools.partial(
        pl.kernel,
        mesh=mesh,
        compiler_params=pltpu.CompilerParams(needs_layout_passes=False),
        out_type=jax.ShapeDtypeStruct((B * D,), jnp.float32),
        scratch_types=[
            pltpu.VMEM((B,), jnp.int32),
            pltpu.VMEM((B + 16,), jnp.int32),
            pltpu.VMEM((B + 16,), jnp.int32),
            pltpu.VMEM((max_ck * _KCAP + 16,), jnp.int32),
            pltpu.VMEM((_NBUF, D, _CH), jnp.float32),
            pltpu.VMEM((_NROWB, D), jnp.float32),
            pltpu.SMEM((max_ck + 8,), jnp.int32),
            pltpu.SemaphoreType.DMA,
            pltpu.SemaphoreType.DMA,
            pltpu.SemaphoreType.DMA,
            pltpu.SemaphoreType.DMA,
            pltpu.SemaphoreType.DMA,
            pltpu.SemaphoreType.DMA,
            pltpu.SemaphoreType.DMA,
            pltpu.SemaphoreType.DMA,
            pltpu.SemaphoreType.DMA,
        ],
    )
    def k(idx_hbm, tT_hbm, tail_hbm, packed_hbm, idx_v, hitpos_v, hitval_v,
          bk_v, chunk_v, rowb_v, cnt_s, sem_r, sem_c0, sem_c1, sem_c2,
          sem_c3, sem_c4, sem_c5, sem_c6, sem_c7):
        wid = lax.axis_index("s") * NC + lax.axis_index("c")
        n_ck = base_ck + jnp.where(wid < n_extra, 1, 0)
        start_ck = wid * base_ck + jnp.minimum(wid, n_extra)
        lo = start_ck * _CH
        hi = lo + n_ck * _CH
        is_last = wid == NW - 1
        sel_hi = jnp.where(is_last, V, hi)
        of_slot = max_ck + 1

        _sems = [sem_c0, sem_c1, sem_c2, sem_c3, sem_c4, sem_c5, sem_c6, sem_c7]
        for b in range(_NBUF):
            pltpu.make_async_copy(
                tT_hbm.at[:, pl.ds(lo + b * _CH, _CH)], chunk_v.at[b],
                _sems[b],
            ).start()

        pltpu.sync_copy(idx_hbm, idx_v)
        iota = lax.iota(jnp.int32, 16)

        def sel_body(g, cnt):
            v = idx_v[pl.ds(g * 16, 16)]
            m = (v >= lo) & (v < sel_hi)
            pos = g * 16 + iota
            plsc.store_compressed(hitpos_v.at[pl.ds(cnt, 16)], pos, mask=m)
            plsc.store_compressed(hitval_v.at[pl.ds(cnt, 16)], v, mask=m)
            return cnt + plsc.all_reduce_population_count(m)[0]

        cnt = lax.fori_loop(0, B // 16, sel_body, jnp.int32(0))
        hitval_v[pl.ds(cnt, 16)] = jnp.full((16,), -1, jnp.int32)
        n_grp = (cnt + 15) // 16

        def zero_body(c, x):
            cnt_s[c] = jnp.int32(0)
            return x
        lax.fori_loop(0, max_ck + 8, zero_body, 0)

        one_lane = iota == 0

        def bk_body(hg, x):
            hv = hitval_v[pl.ds(hg * 16, 16)]
            hp = hitpos_v[pl.ds(hg * 16, 16)]
            mi = ((hv >= lo) & (hv < hi)).astype(jnp.int32)
            for e in range(16):
                @pl.when(mi[e] != 0)
                def _():
                    rel = hv[e] - lo
                    c = rel // _CH
                    slot = cnt_s[c]
                    cnt_s[c] = slot + 1

                    @pl.when(slot < _KCAP)
                    def _():
                        entry = (hp[e] * 1024) + (rel % _CH)
                        plsc.store_compressed(
                            bk_v.at[pl.ds(c * _KCAP + slot, 16)],
                            jnp.broadcast_to(entry, (16,)),
                            mask=one_lane,
                        )

                    @pl.when(slot >= _KCAP)
                    def _():
                        cnt_s[of_slot] = jnp.int32(1)
            return x

        lax.fori_loop(0, n_grp, bk_body, 0)
        overflow = cnt_s[of_slot]

        def drain_one(_, x):
            pltpu.make_async_copy(
                rowb_v.at[0], packed_hbm.at[pl.ds(0, D)], sem_r
            ).wait()
            return x

        def extract(buf, entry, rb):
            pos = entry // 1024
            off = jnp.broadcast_to(entry % 1024, (16,))
            bufi = jnp.broadcast_to(buf, (16,))
            for q in range(D // 16):
                col = plsc.load_gather(
                    chunk_v, [bufi, iota + q * 16, off]
                )
                rowb_v[rb, pl.ds(q * 16, 16)] = col
            pltpu.async_copy(
                rowb_v.at[rb], packed_hbm.at[pl.ds(pos * D, D)], sem_r
            )

        def ck_body(c, n_prev):
            buf = lax.rem(c, _NBUF)
            for b in range(_NBUF):
                @pl.when(buf == b)
                def _(b=b):
                    pltpu.make_async_copy(
                        tT_hbm.at[:, pl.ds(lo + c * _CH, _CH)],
                        chunk_v.at[b],
                        _sems[b],
                    ).wait()

            def fast(x):
                nb = jnp.minimum(cnt_s[c], _KCAP)
                sbase = 0

                def hit_body(t, y):
                    entry = bk_v[pl.ds(c * _KCAP + t, 16)][0]
                    extract(buf, entry, sbase + t)
                    return y

                lax.fori_loop(0, nb, hit_body, 0)
                return nb

            def slow(x):
                off = lo + c * _CH
                sbase = 0

                def grp_body(hg, n_pg):
                    lax.fori_loop(0, n_pg, drain_one, 0)
                    hv = hitval_v[pl.ds(hg * 16, 16)]
                    hp = hitpos_v[pl.ds(hg * 16, 16)]
                    mi = ((hv >= off) & (hv < off + _CH)).astype(jnp.int32)
                    for e in range(16):
                        @pl.when(mi[e] != 0)
                        def _():
                            entry = hp[e] * 1024 + (hv[e] - off)
                            extract(buf, entry, sbase + e)
                    return plsc.all_reduce_population_count(mi != 0)[0]

                return lax.fori_loop(0, n_grp, grp_body, jnp.int32(0))

            nb = lax.cond(overflow == 0, fast, slow, 0)

            for b in range(_NBUF):
                @pl.when((c + _NBUF < n_ck) & (buf == b))
                def _(b=b):
                    pltpu.make_async_copy(
                        tT_hbm.at[:, pl.ds(lo + (c + _NBUF) * _CH, _CH)],
                        chunk_v.at[b],
                        _sems[b],
                    ).start()

            return nb

        def ck_outer(c, n_pending):
            lax.fori_loop(0, n_pending, drain_one, 0)
            return ck_body(c, jnp.int32(0))

        n_pending = lax.fori_loop(0, n_ck, ck_outer, jnp.int32(0))
        lax.fori_loop(0, n_pending, drain_one, 0)

        @pl.when(is_last)
        def _():
            def tail_body(hg, carry):
                hv = hitval_v[pl.ds(hg * 16, 16)]
                hp = hitpos_v[pl.ds(hg * 16, 16)]
                m3i = (hv >= aligned_v).astype(jnp.int32)
                for e in range(16):
                    @pl.when(m3i[e] != 0)
                    def _():
                        pltpu.make_async_copy(
                            tail_hbm.at[pl.ds((hv[e] - aligned_v) * D, D)],
                            rowb_v.at[0],
                            sem_r,
                        ).start()
                        pltpu.make_async_copy(
                            tail_hbm.at[pl.ds(0, D)], rowb_v.at[0], sem_r
                        ).wait()
                        pltpu.async_copy(
                            rowb_v.at[0],
                            packed_hbm.at[pl.ds(hp[e] * D, D)],
                            sem_r,
                        ).wait()
                return carry
            lax.fori_loop(0, n_grp, tail_body, 0)

    return k


def kernel(indices, codes_weight):
    if indices.ndim > 1:
        indices = jnp.squeeze(indices, axis=-1)
    B = indices.shape[0]
    V, D = codes_weight.shape
    info = plsc.get_sparse_core_info()
    NC, NS = info.num_cores, info.num_subcores
    idx = indices.astype(jnp.int32)
    tT = codes_weight.T
    aligned_v = (V // _CH) * _CH
    tail_flat = codes_weight[aligned_v:, :].reshape(-1)
    packed = _phase1_kernel(B, V, D, NC, NS)(idx, tT, tail_flat)
    return packed.reshape(B, D)
